# Initial kernel scaffold; baseline (speedup 1.0000x reference)
import jax
import jax.numpy as jnp
from jax.experimental import pallas as pl


def kernel(a, b, table):
    raise NotImplementedError("write your pallas kernel here")


# speedup vs baseline: 201.6553x; 201.6553x over previous
import functools

import jax
import jax.numpy as jnp
from jax import lax
from jax.experimental import pallas as pl
from jax.experimental.pallas import tpu as pltpu
from jax.experimental.pallas import tpu_sc as plsc

NC, NS, L = 2, 16, 16
NW = NC * NS
TOTAL = 16384 * 200
PER_W = TOTAL // NW
BLK = 12800
NBLK = PER_W // BLK
ITERS = BLK // (2 * L)

_MESH = plsc.VectorSubcoreMesh(
    core_axis_name="c", subcore_axis_name="s", num_cores=NC, num_subcores=NS
)


@functools.partial(
    pl.kernel,
    out_type=jax.ShapeDtypeStruct((TOTAL // 2,), jnp.int32),
    mesh=_MESH,
    scratch_types=[
        pltpu.VMEM((BLK,), jnp.int32),
        pltpu.VMEM((BLK,), jnp.int32),
        pltpu.VMEM((BLK // 2,), jnp.int32),
        pltpu.VMEM((BLK // 2,), jnp.int32),
        pltpu.VMEM((256,), jnp.int32),
        pltpu.SemaphoreType.DMA,
        pltpu.SemaphoreType.DMA,
        pltpu.SemaphoreType.DMA,
        pltpu.SemaphoreType.DMA,
    ],
    compiler_params=pltpu.CompilerParams(needs_layout_passes=False),
)
def _lut_gather(a_hbm, lo_hbm, out_hbm,
                a0, a1, o0, o1, lo_v,
                in_sem0, in_sem1, out_sem0, out_sem1):
    wid = lax.axis_index("s") * NC + lax.axis_index("c")
    base = wid * PER_W
    obase = wid * (PER_W // 2)
    a_bufs = (a0, a1)
    o_bufs = (o0, o1)
    in_sems = (in_sem0, in_sem1)
    out_sems = (out_sem0, out_sem1)

    pltpu.sync_copy(lo_hbm, lo_v)

    def start_in(blk, slot):
        off = base + blk * BLK
        return pltpu.async_copy(
            a_hbm.at[pl.ds(off, BLK)], a_bufs[slot], in_sems[slot]
        )

    def start_out(blk, slot):
        ooff = obase + blk * (BLK // 2)
        return pltpu.async_copy(
            o_bufs[slot], out_hbm.at[pl.ds(ooff, BLK // 2)], out_sems[slot]
        )

    def compute(slot):
        a_ref = a_bufs[slot]
        o_ref = o_bufs[slot]
        iota2 = lax.iota(jnp.int32, L) * 2

        def _body(t, carry):
            b0 = t * (2 * L)
            ev = iota2 + b0
            od = ev + 1
            c_ev = plsc.load_gather(a_ref, [ev]) + 128
            c_od = plsc.load_gather(a_ref, [od]) + 128
            g_lo = plsc.load_gather(lo_v, [c_ev])
            g_hi = plsc.load_gather(lo_v, [c_od]) << 16
            o_ref[pl.ds(t * L, L)] = g_lo | g_hi
            return carry

        lax.fori_loop(0, ITERS, _body, 0)

    in_h = {0: start_in(0, 0)}
    out_h = {}
    for blk in range(NBLK):
        slot = blk % 2
        if blk + 1 < NBLK:
            in_h[blk + 1] = start_in(blk + 1, slot ^ 1)
        in_h[blk].wait()
        if blk >= 2:
            out_h[blk - 2].wait()
        compute(slot)
        out_h[blk] = start_out(blk, slot)
    out_h[NBLK - 2].wait()
    out_h[NBLK - 1].wait()


def kernel(a, b, table):
    idx_b = jnp.asarray(b, jnp.int32) + 128
    column = lax.dynamic_index_in_dim(table, idx_b, axis=1, keepdims=False)
    col32 = column.astype(jnp.int32)
    lo = col32 & 0xFFFF
    a_flat = a.reshape(TOTAL)
    words = _lut_gather(a_flat, lo)
    out = lax.bitcast_convert_type(words, jnp.int16)
    return out.reshape(a.shape)


# speedup vs baseline: 245.7580x; 1.2187x over previous
import functools

import jax
import jax.numpy as jnp
from jax import lax
from jax.experimental import pallas as pl
from jax.experimental.pallas import tpu as pltpu
from jax.experimental.pallas import tpu_sc as plsc

NC, NS, L = 2, 16, 16
NW = NC * NS
TOTAL = 16384 * 200
PER_W = TOTAL // NW
BLK = 12800
NBLK = PER_W // BLK
ITERS = BLK // (2 * L)

_MESH = plsc.VectorSubcoreMesh(
    core_axis_name="c", subcore_axis_name="s", num_cores=NC, num_subcores=NS
)


@functools.partial(
    pl.kernel,
    out_type=jax.ShapeDtypeStruct((TOTAL // 2,), jnp.int32),
    mesh=_MESH,
    scratch_types=[
        pltpu.VMEM((BLK,), jnp.int32),
        pltpu.VMEM((BLK,), jnp.int32),
        pltpu.VMEM((BLK // 2,), jnp.int32),
        pltpu.VMEM((BLK // 2,), jnp.int32),
        pltpu.VMEM((256,), jnp.int32),
        pltpu.SemaphoreType.DMA,
        pltpu.SemaphoreType.DMA,
        pltpu.SemaphoreType.DMA,
        pltpu.SemaphoreType.DMA,
    ],
    compiler_params=pltpu.CompilerParams(needs_layout_passes=False),
)
def _lut_gather(a_hbm, lo_hbm, out_hbm,
                a0, a1, o0, o1, lo_v,
                in_sem0, in_sem1, out_sem0, out_sem1):
    wid = lax.axis_index("s") * NC + lax.axis_index("c")
    base = wid * PER_W
    obase = wid * (PER_W // 2)
    a_bufs = (a0, a1)
    o_bufs = (o0, o1)
    in_sems = (in_sem0, in_sem1)
    out_sems = (out_sem0, out_sem1)

    pltpu.sync_copy(lo_hbm, lo_v)

    def start_in(blk, slot):
        off = base + blk * BLK
        return pltpu.async_copy(
            a_hbm.at[pl.ds(off, BLK)], a_bufs[slot], in_sems[slot]
        )

    def start_out(blk, slot):
        ooff = obase + blk * (BLK // 2)
        return pltpu.async_copy(
            o_bufs[slot], out_hbm.at[pl.ds(ooff, BLK // 2)], out_sems[slot]
        )

    def compute(slot):
        a_ref = a_bufs[slot]
        o_ref = o_bufs[slot]
        iota2 = lax.iota(jnp.int32, L) * 2

        @plsc.parallel_loop(0, ITERS, 1, unroll=8)
        def _body(t):
            b0 = t * (2 * L)
            ev = iota2 + b0
            od = ev + 1
            c_ev = plsc.load_gather(a_ref, [ev]) + 128
            c_od = plsc.load_gather(a_ref, [od]) + 128
            g_lo = plsc.load_gather(lo_v, [c_ev])
            g_hi = plsc.load_gather(lo_v, [c_od]) << 16
            o_ref[pl.ds(t * L, L)] = g_lo | g_hi

    in_h = {0: start_in(0, 0)}
    out_h = {}
    for blk in range(NBLK):
        slot = blk % 2
        if blk + 1 < NBLK:
            in_h[blk + 1] = start_in(blk + 1, slot ^ 1)
        in_h[blk].wait()
        if blk >= 2:
            out_h[blk - 2].wait()
        compute(slot)
        out_h[blk] = start_out(blk, slot)
    out_h[NBLK - 2].wait()
    out_h[NBLK - 1].wait()


def kernel(a, b, table):
    idx_b = jnp.asarray(b, jnp.int32) + 128
    column = lax.dynamic_index_in_dim(table, idx_b, axis=1, keepdims=False)
    col32 = column.astype(jnp.int32)
    lo = col32 & 0xFFFF
    a_flat = a.reshape(TOTAL)
    words = _lut_gather(a_flat, lo)
    out = lax.bitcast_convert_type(words, jnp.int16)
    return out.reshape(a.shape)
